# baseline (speedup 1.0000x reference)
import functools

import jax
import jax.numpy as jnp
from jax import lax
from jax.experimental import pallas as pl
from jax.experimental.pallas import tpu as pltpu
from jax.experimental.pallas import tpu_sc as plsc

N_NODES = 50000
N_EDGES = 800000

NC = 2
NS = 16
CHUNK = 128
G_CHUNKS = 392
E_PAD = NS * G_CHUNKS * CHUNK
N_PAD = 50176
ROWS_PER_SUB = N_PAD // NS
NQ = 4
TRASH = N_NODES

E_TILE = 2048

_SC_MESH = plsc.VectorSubcoreMesh(core_axis_name="c", subcore_axis_name="s",
                                  num_cores=NC, num_subcores=NS)
_SC_PARAMS = pltpu.CompilerParams(use_tc_tiling_on_sc=False)


def _edge_u_body(ea_ref, w1t_ref, b1_ref, w2t_ref, c2_ref,
                 o1_ref, o2_ref, o3_ref):
    ea = ea_ref[...]
    outs = (o1_ref, o2_ref, o3_ref)
    for l in range(3):
        t = ea[:, 0:1] * w1t_ref[l, 0:1, :]
        t += ea[:, 1:2] * w1t_ref[l, 1:2, :]
        t += ea[:, 2:3] * w1t_ref[l, 2:3, :]
        t = jnp.maximum(t + b1_ref[l], 0.0)
        u = jnp.dot(t, w2t_ref[l], preferred_element_type=jnp.float32)
        u = u + c2_ref[l]
        for q in range(NQ):
            outs[l][q] = u[:, 16 * q:16 * q + 16]


def _edge_u(ea_pad, w1t, b1, w2t, c2):
    out_sds = jax.ShapeDtypeStruct((NQ, E_PAD, 16), jnp.float32)
    full = lambda *s: pl.BlockSpec(s, lambda i: tuple(0 for _ in s))
    return pl.pallas_call(
        _edge_u_body,
        grid=(E_PAD // E_TILE,),
        in_specs=[
            pl.BlockSpec((E_TILE, 3), lambda i: (i, 0)),
            full(3, 3, 64),
            full(3, 1, 64),
            full(3, 64, 64),
            full(3, 1, 64),
        ],
        out_specs=[pl.BlockSpec((NQ, E_TILE, 16), lambda i: (0, i, 0))] * 3,
        out_shape=[out_sds] * 3,
    )(ea_pad, w1t, b1, w2t, c2)


def _bcast_lane(vec, i):
    idx = jnp.full((16, 1), i, jnp.int32)
    dnums = lax.GatherDimensionNumbers(offset_dims=(),
                                       collapsed_slice_dims=(0,),
                                       start_index_map=(0,))
    return lax.gather(vec, idx, dnums, slice_sizes=(1,),
                      mode=lax.GatherScatterMode.PROMISE_IN_BOUNDS)


def _gather_scatter_body(relu_add, srcr, idxo, same, tab, u6, zeros_hbm,
                         out_hbm, idxs_v, idxo_v, sm_v, rows_v, u_v, acc_sh,
                         sem):
    c = lax.axis_index("c")
    s = lax.axis_index("s")
    sl = pl.ds(s * ROWS_PER_SUB, ROWS_PER_SUB)
    for q in range(2):
        k = c * 2 + q
        pltpu.sync_copy(zeros_hbm.at[sl], acc_sh.at[sl])
        plsc.subcore_barrier()

        def body(g, carry):
            pltpu.sync_copy(srcr.at[k, s, g], idxs_v)
            pltpu.sync_copy(idxo.at[s, g], idxo_v)
            pltpu.sync_copy(same.at[s, g], sm_v)
            pltpu.async_copy(tab.at[idxs_v], rows_v, sem).wait()
            if relu_add:
                pltpu.sync_copy(u6.at[k, s, g], u_v)
            acc = jnp.zeros((16,), jnp.float32)
            for blk in range(CHUNK // 16):
                smrow = sm_v[pl.ds(blk * 16, 16)]
                for i in range(16):
                    j = blk * 16 + i
                    if relu_add:
                        w = jnp.maximum(rows_v[j] + u_v[j], 0.0)
                    else:
                        w = rows_v[j]
                    acc = w + _bcast_lane(smrow, i) * acc
                    rows_v[j] = acc
            pltpu.sync_copy(rows_v, acc_sh.at[idxo_v], add=True)
            return carry

        lax.fori_loop(0, G_CHUNKS, body, 0)
        plsc.subcore_barrier()
        pltpu.sync_copy(acc_sh.at[sl], out_hbm.at[k, sl])
        plsc.subcore_barrier()


def _make_gs_kernel(relu_add):
    scratch = [
        pltpu.VMEM((CHUNK,), jnp.int32),
        pltpu.VMEM((CHUNK,), jnp.int32),
        pltpu.VMEM((CHUNK,), jnp.float32),
        pltpu.VMEM((CHUNK, 16), jnp.float32),
        pltpu.VMEM((CHUNK, 16), jnp.float32),
        pltpu.VMEM_SHARED((N_PAD, 16), jnp.float32),
        pltpu.SemaphoreType.DMA,
    ]

    @functools.partial(
        pl.kernel,
        out_type=jax.ShapeDtypeStruct((NQ, N_PAD, 16), jnp.float32),
        mesh=_SC_MESH,
        compiler_params=_SC_PARAMS,
        scratch_types=scratch,
    )
    def k(srcr, idxo, same, tab, u6, zeros_hbm, out_hbm,
          idxs_v, idxo_v, sm_v, rows_v, u_v, acc_sh, sem):
        _gather_scatter_body(relu_add, srcr, idxo, same, tab, u6, zeros_hbm,
                             out_hbm, idxs_v, idxo_v, sm_v, rows_v, u_v,
                             acc_sh, sem)

    return k


_sc_radar = _make_gs_kernel(True)
_sc_gcn = _make_gs_kernel(False)


def _to_quarters(a):
    return jnp.concatenate([a[:, 16 * q:16 * q + 16] for q in range(NQ)])


def _from_quarters(o):
    return jnp.concatenate([o[q] for q in range(NQ)], axis=-1)


def _lin(x, w, b):
    return x @ w.T + b


def _mlp(x, w1, b1, w2, b2):
    return _lin(jax.nn.relu(_lin(x, w1, b1)), w2, b2)


def kernel(x, edge_index, edge_attr, params):
    sp = params['sp']
    cl = params['cl']

    src = edge_index[0]
    dst = edge_index[1]
    perm = jnp.argsort(dst)
    dst_s = dst[perm]
    src_s = src[perm]
    ea_s = edge_attr[perm]
    npad = E_PAD - N_EDGES
    dst_sp = jnp.concatenate([dst_s, jnp.full((npad,), TRASH, jnp.int32)])
    src_sp = jnp.concatenate([src_s, jnp.full((npad,), N_NODES, jnp.int32)])
    pos = jnp.arange(E_PAD, dtype=jnp.int32)
    dprev = jnp.concatenate([jnp.full((1,), -1, jnp.int32), dst_sp[:-1]])
    dnext = jnp.concatenate([dst_sp[1:], jnp.full((1,), -2, jnp.int32)])
    same = ((pos % CHUNK != 0) & (dst_sp == dprev)).astype(jnp.float32)
    last = (pos % CHUNK == CHUNK - 1) | (dst_sp != dnext)
    idxo = jnp.where(last & (dst_sp < N_NODES), dst_sp, TRASH + pos % CHUNK)
    srcr = jnp.stack([src_sp + q * N_PAD for q in range(NQ)])
    srcr = srcr.reshape(NQ, NS, G_CHUNKS, CHUNK)
    idxo = idxo.reshape(NS, G_CHUNKS, CHUNK)
    same = same.reshape(NS, G_CHUNKS, CHUNK)
    zeros16 = jnp.zeros((N_PAD, 16), jnp.float32)
    x_pad = jnp.pad(x, ((0, N_PAD - N_NODES), (0, 0)))
    ea_pad = jnp.pad(ea_s, ((0, npad), (0, 0)))

    w1t = jnp.stack([sp['convs'][l]['ew1'].T for l in range(3)])
    b1 = jnp.stack([sp['convs'][l]['eb1'][None, :] for l in range(3)])
    w2t = jnp.stack([(sp['convs'][l]['mw1'][:, 64:] @ sp['convs'][l]['ew2']).T
                     for l in range(3)])
    c2 = jnp.stack([(sp['convs'][l]['eb2'] @ sp['convs'][l]['mw1'][:, 64:].T
                     + sp['convs'][l]['mb1'])[None, :] for l in range(3)])
    u123 = _edge_u(ea_pad, w1t, b1, w2t, c2)
    u6s = [u.reshape(NQ, NS, G_CHUNKS, CHUNK, 16) for u in u123]

    bounds = jnp.searchsorted(dst_s, jnp.arange(N_NODES + 1, dtype=jnp.int32))
    cnt = (bounds[1:] - bounds[:-1]).astype(jnp.float32)
    cntm = jnp.maximum(cnt, 1.0)
    has = (cnt > 0).astype(jnp.float32)
    dinv_n = (cnt + 1.0) ** -0.5
    dinv = jnp.pad(dinv_n, (0, N_PAD - N_NODES))

    h = _lin(x_pad, sp['inp_w'], sp['inp_b'])
    for l in range(3):
        p = sp['convs'][l]
        xn = _mlp(h, p['nw1'], p['nb1'], p['nw2'], p['nb2'])
        a_tab = _to_quarters(xn @ p['mw1'][:, :64].T)
        rsum = _from_quarters(_sc_radar(srcr, idxo, same, a_tab, u6s[l],
                                        zeros16))
        rsum = rsum[:N_NODES]
        mean = (rsum / cntm[:, None]) @ p['mw2'].T + has[:, None] * p['mb2']
        o = mean + xn[:N_NODES]
        bn = sp['bns'][l]
        o = o * (bn['g'] / jnp.sqrt(1.0 + 1e-5)) + bn['b']
        h = h.at[:N_NODES].add(jax.nn.relu(o))
    spatial = _lin(h, sp['out_w'], sp['out_b'])

    x2 = spatial
    for i in range(3):
        g = cl['gcn'][i]
        y = (x2 @ g['w'].T) * dinv[:, None]
        y_tab = _to_quarters(y)
        acc = _from_quarters(_sc_gcn(srcr, idxo, same, y_tab, u6s[0],
                                     zeros16))
        xn2 = jax.nn.relu(dinv[:, None] * (acc + y) + g['b'])
        x2 = x2 + xn2 if i > 0 else xn2

    att = jax.nn.sigmoid(_lin(jax.nn.relu(_lin(x2, cl['att_w1'], cl['att_b1'])),
                              cl['att_w2'], cl['att_b2']))
    x2 = x2 * att
    logits = _lin(jax.nn.relu(_lin(x2, cl['cls_w1'], cl['cls_b1'])),
                  cl['cls_w2'], cl['cls_b2'])
    return spatial[:N_NODES], logits[:N_NODES]


# speedup vs baseline: 1.6421x; 1.6421x over previous
import functools

import jax
import jax.numpy as jnp
from jax import lax
from jax.experimental import pallas as pl
from jax.experimental.pallas import tpu as pltpu
from jax.experimental.pallas import tpu_sc as plsc

N_NODES = 50000
N_EDGES = 800000

NC = 2
NS = 16
CHUNK = 128
G_CHUNKS = 392
E_PAD = NS * G_CHUNKS * CHUNK
N_PAD = 50176
ROWS_PER_SUB = N_PAD // NS
NQ = 4
TRASH = N_NODES

E_TILE = 2048

_SC_MESH = plsc.VectorSubcoreMesh(core_axis_name="c", subcore_axis_name="s",
                                  num_cores=NC, num_subcores=NS)
_SC_PARAMS = pltpu.CompilerParams(use_tc_tiling_on_sc=False)


def _edge_u_body(ea_ref, w1t_ref, b1_ref, w2t_ref, c2_ref,
                 o1_ref, o2_ref, o3_ref):
    ea = ea_ref[...]
    outs = (o1_ref, o2_ref, o3_ref)
    for l in range(3):
        t = ea[:, 0:1] * w1t_ref[l, 0:1, :]
        t += ea[:, 1:2] * w1t_ref[l, 1:2, :]
        t += ea[:, 2:3] * w1t_ref[l, 2:3, :]
        t = jnp.maximum(t + b1_ref[l], 0.0)
        u = jnp.dot(t, w2t_ref[l], preferred_element_type=jnp.float32)
        u = u + c2_ref[l]
        outs[l][...] = jnp.concatenate([u, jnp.zeros_like(u)], axis=1)


def _edge_u(ea_pad, w1t, b1, w2t, c2):
    out_sds = jax.ShapeDtypeStruct((E_PAD, 128), jnp.float32)
    full = lambda *s: pl.BlockSpec(s, lambda i: tuple(0 for _ in s))
    return pl.pallas_call(
        _edge_u_body,
        grid=(E_PAD // E_TILE,),
        in_specs=[
            pl.BlockSpec((E_TILE, 3), lambda i: (i, 0)),
            full(3, 3, 64),
            full(3, 1, 64),
            full(3, 64, 64),
            full(3, 1, 64),
        ],
        out_specs=[pl.BlockSpec((E_TILE, 128), lambda i: (i, 0))] * 3,
        out_shape=[out_sds] * 3,
    )(ea_pad, w1t, b1, w2t, c2)


def _bcast_lane(vec, i):
    idx = jnp.full((16, 1), i, jnp.int32)
    dnums = lax.GatherDimensionNumbers(offset_dims=(),
                                       collapsed_slice_dims=(0,),
                                       start_index_map=(0,))
    return lax.gather(vec, idx, dnums, slice_sizes=(1,),
                      mode=lax.GatherScatterMode.PROMISE_IN_BOUNDS)


NPAIR = G_CHUNKS // 2


def _gather_scatter_body(relu_add, meta, tab, u2d, zeros_hbm, out_hbm,
                         meta_v, idx2_v, rows2_v, u2_v, acc_sh,
                         msem, gsem0, gsem1, usem0, usem1):
    c = lax.axis_index("c")
    s = lax.axis_index("s")
    sl = pl.ds(s * ROWS_PER_SUB, ROWS_PER_SUB)
    gsems = (gsem0, gsem1)
    usems = (usem0, usem1)
    for q in range(2):
        k = c * 2 + q
        koff = k * N_PAD
        pltpu.sync_copy(zeros_hbm.at[sl], acc_sh.at[sl])
        plsc.subcore_barrier()

        def body(i, carry):
            md = pltpu.async_copy(meta.at[s, i], meta_v, msem)
            uds = []
            if relu_add:
                for h in range(2):
                    ebase = ((s * G_CHUNKS + 2 * i + h)) * CHUNK
                    uds.append(pltpu.async_copy(
                        u2d.at[pl.ds(ebase, CHUNK), pl.ds(16 * k, 16)],
                        u2_v.at[h], usems[h]))
            md.wait()
            gds = []
            for h in range(2):
                for blk in range(CHUNK // 16):
                    slc = pl.ds(blk * 16, 16)
                    idx2_v[h, slc] = meta_v[h, 0, slc] + koff
                gds.append(pltpu.async_copy(tab.at[idx2_v.at[h]],
                                            rows2_v.at[h], gsems[h]))
            for h in range(2):
                gds[h].wait()
                if relu_add:
                    uds[h].wait()
                acc = jnp.zeros((16,), jnp.float32)
                for blk in range(CHUNK // 16):
                    smrow = lax.bitcast_convert_type(
                        meta_v[h, 2, pl.ds(blk * 16, 16)], jnp.float32)
                    for i2 in range(16):
                        j = blk * 16 + i2
                        if relu_add:
                            w = jnp.maximum(rows2_v[h, j] + u2_v[h, j], 0.0)
                        else:
                            w = rows2_v[h, j]
                        acc = w + _bcast_lane(smrow, i2) * acc
                        rows2_v[h, j] = acc
                pltpu.sync_copy(rows2_v.at[h], acc_sh.at[meta_v.at[h, 1]],
                                add=True)
            return carry

        lax.fori_loop(0, NPAIR, body, 0)
        plsc.subcore_barrier()
        pltpu.sync_copy(acc_sh.at[sl], out_hbm.at[k, sl])
        plsc.subcore_barrier()


def _make_gs_kernel(relu_add):
    scratch = [
        pltpu.VMEM((2, 3, CHUNK), jnp.int32),
        pltpu.VMEM((2, CHUNK), jnp.int32),
        pltpu.VMEM((2, CHUNK, 16), jnp.float32),
        pltpu.VMEM((2, CHUNK, 16), jnp.float32),
        pltpu.VMEM_SHARED((N_PAD, 16), jnp.float32),
        pltpu.SemaphoreType.DMA,
        pltpu.SemaphoreType.DMA,
        pltpu.SemaphoreType.DMA,
        pltpu.SemaphoreType.DMA,
        pltpu.SemaphoreType.DMA,
    ]

    @functools.partial(
        pl.kernel,
        out_type=jax.ShapeDtypeStruct((NQ, N_PAD, 16), jnp.float32),
        mesh=_SC_MESH,
        compiler_params=_SC_PARAMS,
        scratch_types=scratch,
    )
    def k(meta, tab, u2d, zeros_hbm, out_hbm,
          meta_v, idx2_v, rows2_v, u2_v, acc_sh,
          msem, gsem0, gsem1, usem0, usem1):
        _gather_scatter_body(relu_add, meta, tab, u2d, zeros_hbm, out_hbm,
                             meta_v, idx2_v, rows2_v, u2_v, acc_sh,
                             msem, gsem0, gsem1, usem0, usem1)

    return k


_sc_radar = _make_gs_kernel(True)
_sc_gcn = _make_gs_kernel(False)


def _to_quarters(a):
    return jnp.concatenate([a[:, 16 * q:16 * q + 16] for q in range(NQ)])


def _from_quarters(o):
    return jnp.concatenate([o[q] for q in range(NQ)], axis=-1)


def _lin(x, w, b):
    return x @ w.T + b


def _mlp(x, w1, b1, w2, b2):
    return _lin(jax.nn.relu(_lin(x, w1, b1)), w2, b2)


def kernel(x, edge_index, edge_attr, params):
    sp = params['sp']
    cl = params['cl']

    src = edge_index[0]
    dst = edge_index[1]
    perm = jnp.argsort(dst)
    dst_s = dst[perm]
    src_s = src[perm]
    ea_s = edge_attr[perm]
    npad = E_PAD - N_EDGES
    dst_sp = jnp.concatenate([dst_s, jnp.full((npad,), TRASH, jnp.int32)])
    src_sp = jnp.concatenate([src_s, jnp.full((npad,), N_NODES, jnp.int32)])
    pos = jnp.arange(E_PAD, dtype=jnp.int32)
    dprev = jnp.concatenate([jnp.full((1,), -1, jnp.int32), dst_sp[:-1]])
    dnext = jnp.concatenate([dst_sp[1:], jnp.full((1,), -2, jnp.int32)])
    same = ((pos % CHUNK != 0) & (dst_sp == dprev)).astype(jnp.float32)
    last = (pos % CHUNK == CHUNK - 1) | (dst_sp != dnext)
    idxo = jnp.where(last & (dst_sp < N_NODES), dst_sp, TRASH + pos % CHUNK)
    same_bits = lax.bitcast_convert_type(same, jnp.int32)
    meta = jnp.stack([src_sp, idxo, same_bits])
    meta = meta.reshape(3, NS, G_CHUNKS, CHUNK).transpose(1, 2, 0, 3)
    meta = meta.reshape(NS, NPAIR, 2, 3, CHUNK)
    zeros16 = jnp.zeros((N_PAD, 16), jnp.float32)
    x_pad = jnp.pad(x, ((0, N_PAD - N_NODES), (0, 0)))
    ea_pad = jnp.pad(ea_s, ((0, npad), (0, 0)))

    w1t = jnp.stack([sp['convs'][l]['ew1'].T for l in range(3)])
    b1 = jnp.stack([sp['convs'][l]['eb1'][None, :] for l in range(3)])
    w2t = jnp.stack([(sp['convs'][l]['mw1'][:, 64:] @ sp['convs'][l]['ew2']).T
                     for l in range(3)])
    c2 = jnp.stack([(sp['convs'][l]['eb2'] @ sp['convs'][l]['mw1'][:, 64:].T
                     + sp['convs'][l]['mb1'])[None, :] for l in range(3)])
    u123 = _edge_u(ea_pad, w1t, b1, w2t, c2)

    bounds = jnp.searchsorted(dst_s, jnp.arange(N_NODES + 1, dtype=jnp.int32))
    cnt = (bounds[1:] - bounds[:-1]).astype(jnp.float32)
    cntm = jnp.maximum(cnt, 1.0)
    has = (cnt > 0).astype(jnp.float32)
    dinv_n = (cnt + 1.0) ** -0.5
    dinv = jnp.pad(dinv_n, (0, N_PAD - N_NODES))

    h = _lin(x_pad, sp['inp_w'], sp['inp_b'])
    for l in range(3):
        p = sp['convs'][l]
        xn = _mlp(h, p['nw1'], p['nb1'], p['nw2'], p['nb2'])
        a_tab = _to_quarters(xn @ p['mw1'][:, :64].T)
        rsum = _from_quarters(_sc_radar(meta, a_tab, u123[l], zeros16))
        rsum = rsum[:N_NODES]
        mean = (rsum / cntm[:, None]) @ p['mw2'].T + has[:, None] * p['mb2']
        o = mean + xn[:N_NODES]
        bn = sp['bns'][l]
        o = o * (bn['g'] / jnp.sqrt(1.0 + 1e-5)) + bn['b']
        h = h.at[:N_NODES].add(jax.nn.relu(o))
    spatial = _lin(h, sp['out_w'], sp['out_b'])

    x2 = spatial
    for i in range(3):
        g = cl['gcn'][i]
        y = (x2 @ g['w'].T) * dinv[:, None]
        y_tab = _to_quarters(y)
        acc = _from_quarters(_sc_gcn(meta, y_tab, u123[0], zeros16))
        xn2 = jax.nn.relu(dinv[:, None] * (acc + y) + g['b'])
        x2 = x2 + xn2 if i > 0 else xn2

    att = jax.nn.sigmoid(_lin(jax.nn.relu(_lin(x2, cl['att_w1'], cl['att_b1'])),
                              cl['att_w2'], cl['att_b2']))
    x2 = x2 * att
    logits = _lin(jax.nn.relu(_lin(x2, cl['cls_w1'], cl['cls_b1'])),
                  cl['cls_w2'], cl['cls_b2'])
    return spatial[:N_NODES], logits[:N_NODES]


# speedup vs baseline: 1.6642x; 1.0134x over previous
import functools

import jax
import jax.numpy as jnp
from jax import lax
from jax.experimental import pallas as pl
from jax.experimental.pallas import tpu as pltpu
from jax.experimental.pallas import tpu_sc as plsc

N_NODES = 50000
N_EDGES = 800000

NC = 2
NS = 16
CHUNK = 128
G_CHUNKS = 392
E_PAD = NS * G_CHUNKS * CHUNK
N_PAD = 50176
ROWS_PER_SUB = N_PAD // NS
NQ = 4
TRASH = N_NODES

E_TILE = 2048

_SC_MESH = plsc.VectorSubcoreMesh(core_axis_name="c", subcore_axis_name="s",
                                  num_cores=NC, num_subcores=NS)
_SC_PARAMS = pltpu.CompilerParams(use_tc_tiling_on_sc=False)


def _edge_u_body(ea_ref, w1t_ref, b1_ref, w2t_ref, c2_ref,
                 o1_ref, o2_ref, o3_ref):
    ea = ea_ref[...]
    outs = (o1_ref, o2_ref, o3_ref)
    for l in range(3):
        t = ea[:, 0:1] * w1t_ref[l, 0:1, :]
        t += ea[:, 1:2] * w1t_ref[l, 1:2, :]
        t += ea[:, 2:3] * w1t_ref[l, 2:3, :]
        t = jnp.maximum(t + b1_ref[l], 0.0)
        u = jnp.dot(t, w2t_ref[l], preferred_element_type=jnp.float32)
        u = u + c2_ref[l]
        outs[l][...] = jnp.concatenate([u, jnp.zeros_like(u)], axis=1)


def _edge_u(ea_pad, w1t, b1, w2t, c2):
    out_sds = jax.ShapeDtypeStruct((E_PAD, 128), jnp.float32)
    full = lambda *s: pl.BlockSpec(s, lambda i: tuple(0 for _ in s))
    return pl.pallas_call(
        _edge_u_body,
        grid=(E_PAD // E_TILE,),
        in_specs=[
            pl.BlockSpec((E_TILE, 3), lambda i: (i, 0)),
            full(3, 3, 64),
            full(3, 1, 64),
            full(3, 64, 64),
            full(3, 1, 64),
        ],
        out_specs=[pl.BlockSpec((E_TILE, 128), lambda i: (i, 0))] * 3,
        out_shape=[out_sds] * 3,
    )(ea_pad, w1t, b1, w2t, c2)


def _bcast_lane(vec, i):
    idx = jnp.full((16, 1), i, jnp.int32)
    dnums = lax.GatherDimensionNumbers(offset_dims=(),
                                       collapsed_slice_dims=(0,),
                                       start_index_map=(0,))
    return lax.gather(vec, idx, dnums, slice_sizes=(1,),
                      mode=lax.GatherScatterMode.PROMISE_IN_BOUNDS)


NPAIR = G_CHUNKS // 2


def _gather_scatter_body(relu_add, meta, tab, u2d, zeros_hbm, out_hbm,
                         meta_v, idx2_v, rows2_v, u2_v, acc_sh,
                         msem, gsem0, gsem1, usem0, usem1):
    c = lax.axis_index("c")
    s = lax.axis_index("s")
    sl = pl.ds(s * ROWS_PER_SUB, ROWS_PER_SUB)
    gsems = (gsem0, gsem1)
    usems = (usem0, usem1)
    for q in range(2):
        k = c * 2 + q
        koff = k * N_PAD
        pltpu.sync_copy(zeros_hbm.at[sl], acc_sh.at[sl])
        plsc.subcore_barrier()

        def body(i, carry):
            md = pltpu.async_copy(meta.at[s, i], meta_v, msem)
            uds = []
            if relu_add:
                for h in range(2):
                    ebase = ((s * G_CHUNKS + 2 * i + h)) * CHUNK
                    uds.append(pltpu.async_copy(
                        u2d.at[pl.ds(ebase, CHUNK), pl.ds(16 * k, 16)],
                        u2_v.at[h], usems[h]))
            md.wait()
            gds = []
            for h in range(2):
                for blk in range(CHUNK // 16):
                    slc = pl.ds(blk * 16, 16)
                    idx2_v[h, slc] = meta_v[h, 0, slc] + koff
                gds.append(pltpu.async_copy(tab.at[idx2_v.at[h]],
                                            rows2_v.at[h], gsems[h]))
            for h in range(2):
                gds[h].wait()
                if relu_add:
                    uds[h].wait()
                acc = jnp.zeros((16,), jnp.float32)
                for blk in range(CHUNK // 16):
                    smrow = lax.bitcast_convert_type(
                        meta_v[h, 2, pl.ds(blk * 16, 16)], jnp.float32)
                    for i2 in range(16):
                        j = blk * 16 + i2
                        if relu_add:
                            w = jnp.maximum(rows2_v[h, j] + u2_v[h, j], 0.0)
                        else:
                            w = rows2_v[h, j]
                        acc = w + _bcast_lane(smrow, i2) * acc
                        rows2_v[h, j] = acc
                pltpu.sync_copy(rows2_v.at[h], acc_sh.at[meta_v.at[h, 1]],
                                add=True)
            return carry

        lax.fori_loop(0, NPAIR, body, 0)
        plsc.subcore_barrier()
        pltpu.sync_copy(acc_sh.at[sl], out_hbm.at[k, sl])
        plsc.subcore_barrier()


def _make_gs_kernel(relu_add):
    scratch = [
        pltpu.VMEM((2, 3, CHUNK), jnp.int32),
        pltpu.VMEM((2, CHUNK), jnp.int32),
        pltpu.VMEM((2, CHUNK, 16), jnp.float32),
        pltpu.VMEM((2, CHUNK, 16), jnp.float32),
        pltpu.VMEM_SHARED((N_PAD, 16), jnp.float32),
        pltpu.SemaphoreType.DMA,
        pltpu.SemaphoreType.DMA,
        pltpu.SemaphoreType.DMA,
        pltpu.SemaphoreType.DMA,
        pltpu.SemaphoreType.DMA,
    ]

    @functools.partial(
        pl.kernel,
        out_type=jax.ShapeDtypeStruct((NQ, N_PAD, 16), jnp.float32),
        mesh=_SC_MESH,
        compiler_params=_SC_PARAMS,
        scratch_types=scratch,
    )
    def k(meta, tab, u2d, zeros_hbm, out_hbm,
          meta_v, idx2_v, rows2_v, u2_v, acc_sh,
          msem, gsem0, gsem1, usem0, usem1):
        _gather_scatter_body(relu_add, meta, tab, u2d, zeros_hbm, out_hbm,
                             meta_v, idx2_v, rows2_v, u2_v, acc_sh,
                             msem, gsem0, gsem1, usem0, usem1)

    return k


_sc_radar = _make_gs_kernel(True)
_sc_gcn = _make_gs_kernel(False)


def _to_quarters(a):
    return jnp.concatenate([a[:, 16 * q:16 * q + 16] for q in range(NQ)])


def _from_quarters(o):
    return jnp.concatenate([o[q] for q in range(NQ)], axis=-1)


def _lin(x, w, b):
    return x @ w.T + b


def _mlp(x, w1, b1, w2, b2):
    return _lin(jax.nn.relu(_lin(x, w1, b1)), w2, b2)


def kernel(x, edge_index, edge_attr, params):
    sp = params['sp']
    cl = params['cl']

    src = edge_index[0]
    dst = edge_index[1]
    perm = jnp.argsort(dst, stable=False)
    dst_s = dst[perm]
    src_s = src[perm]
    ea_s = edge_attr[perm]
    npad = E_PAD - N_EDGES
    dst_sp = jnp.concatenate([dst_s, jnp.full((npad,), TRASH, jnp.int32)])
    src_sp = jnp.concatenate([src_s, jnp.full((npad,), N_NODES, jnp.int32)])
    pos = jnp.arange(E_PAD, dtype=jnp.int32)
    dprev = jnp.concatenate([jnp.full((1,), -1, jnp.int32), dst_sp[:-1]])
    dnext = jnp.concatenate([dst_sp[1:], jnp.full((1,), -2, jnp.int32)])
    same = ((pos % CHUNK != 0) & (dst_sp == dprev)).astype(jnp.float32)
    last = (pos % CHUNK == CHUNK - 1) | (dst_sp != dnext)
    idxo = jnp.where(last & (dst_sp < N_NODES), dst_sp, TRASH + pos % CHUNK)
    same_bits = lax.bitcast_convert_type(same, jnp.int32)
    meta = jnp.stack([src_sp, idxo, same_bits])
    meta = meta.reshape(3, NS, G_CHUNKS, CHUNK).transpose(1, 2, 0, 3)
    meta = meta.reshape(NS, NPAIR, 2, 3, CHUNK)
    zeros16 = jnp.zeros((N_PAD, 16), jnp.float32)
    x_pad = jnp.pad(x, ((0, N_PAD - N_NODES), (0, 0)))
    ea_pad = jnp.pad(ea_s, ((0, npad), (0, 0)))

    w1t = jnp.stack([sp['convs'][l]['ew1'].T for l in range(3)])
    b1 = jnp.stack([sp['convs'][l]['eb1'][None, :] for l in range(3)])
    w2t = jnp.stack([(sp['convs'][l]['mw1'][:, 64:] @ sp['convs'][l]['ew2']).T
                     for l in range(3)])
    c2 = jnp.stack([(sp['convs'][l]['eb2'] @ sp['convs'][l]['mw1'][:, 64:].T
                     + sp['convs'][l]['mb1'])[None, :] for l in range(3)])
    u123 = _edge_u(ea_pad, w1t, b1, w2t, c2)

    bounds = jnp.searchsorted(dst_s, jnp.arange(N_NODES + 1, dtype=jnp.int32))
    cnt = (bounds[1:] - bounds[:-1]).astype(jnp.float32)
    cntm = jnp.maximum(cnt, 1.0)
    has = (cnt > 0).astype(jnp.float32)
    dinv_n = (cnt + 1.0) ** -0.5
    dinv = jnp.pad(dinv_n, (0, N_PAD - N_NODES))

    h = _lin(x_pad, sp['inp_w'], sp['inp_b'])
    for l in range(3):
        p = sp['convs'][l]
        xn = _mlp(h, p['nw1'], p['nb1'], p['nw2'], p['nb2'])
        a_tab = _to_quarters(xn @ p['mw1'][:, :64].T)
        rsum = _from_quarters(_sc_radar(meta, a_tab, u123[l], zeros16))
        rsum = rsum[:N_NODES]
        mean = (rsum / cntm[:, None]) @ p['mw2'].T + has[:, None] * p['mb2']
        o = mean + xn[:N_NODES]
        bn = sp['bns'][l]
        o = o * (bn['g'] / jnp.sqrt(1.0 + 1e-5)) + bn['b']
        h = h.at[:N_NODES].add(jax.nn.relu(o))
    spatial = _lin(h, sp['out_w'], sp['out_b'])

    x2 = spatial
    for i in range(3):
        g = cl['gcn'][i]
        y = (x2 @ g['w'].T) * dinv[:, None]
        y_tab = _to_quarters(y)
        acc = _from_quarters(_sc_gcn(meta, y_tab, u123[0], zeros16))
        xn2 = jax.nn.relu(dinv[:, None] * (acc + y) + g['b'])
        x2 = x2 + xn2 if i > 0 else xn2

    att = jax.nn.sigmoid(_lin(jax.nn.relu(_lin(x2, cl['att_w1'], cl['att_b1'])),
                              cl['att_w2'], cl['att_b2']))
    x2 = x2 * att
    logits = _lin(jax.nn.relu(_lin(x2, cl['cls_w1'], cl['cls_b1'])),
                  cl['cls_w2'], cl['cls_b2'])
    return spatial[:N_NODES], logits[:N_NODES]
